# baseline (speedup 1.0000x reference)
import functools

import jax
import jax.numpy as jnp
from jax import lax
from jax.experimental import pallas as pl
from jax.experimental.pallas import tpu as pltpu
from jax.experimental.pallas import tpu_sc as plsc

_NC = 2
_NS = 16
_NW = _NC * _NS
_L = 16
_CH = 256


@functools.lru_cache(maxsize=None)
def _make_gather(B: int, D: int):
    assert B % (8 * _NW) == 0
    bpw = B // _NW
    D2 = 2 * D

    mesh = plsc.VectorSubcoreMesh(core_axis_name="c", subcore_axis_name="s")

    @functools.partial(
        pl.kernel,
        out_type=jax.ShapeDtypeStruct((B // 2, 2 * D), jnp.float32),
        mesh=mesh,
        scratch_types=[
            pltpu.VMEM((bpw,), jnp.int32),
            pltpu.VMEM((bpw,), jnp.int32),
            pltpu.VMEM((_CH, D2), jnp.float32),
            pltpu.VMEM((bpw // 2, D2), jnp.float32),
            pltpu.SemaphoreType.DMA,
        ],
        compiler_params=pltpu.CompilerParams(
            use_tc_tiling_on_sc=True, needs_layout_passes=False),
    )
    def gather(table2_hbm, idx_hbm, out_hbm, ids_v, rows_i, buf_v, out_v, sem):
        wid = lax.axis_index("s") * _NC + lax.axis_index("c")
        base = pl.multiple_of(wid * bpw, bpw)
        pltpu.sync_copy(idx_hbm.at[pl.ds(base, bpw)], ids_v)

        def mk_rows(k, _):
            v = ids_v[pl.ds(k * _L, _L)]
            rows_i[pl.ds(k * _L, _L)] = jax.lax.shift_right_logical(v, 1)
            return _
        lax.fori_loop(0, bpw // _L, mk_rows, 0, unroll=4)

        for c in range(bpw // _CH):
            pltpu.async_copy(
                table2_hbm.at[rows_i.at[pl.ds(c * _CH, _CH)]],
                buf_v, sem).wait()

            def extract(k, _, c=c):
                j = lax.iota(jnp.int32, _L) + k * _L
                ids16 = ids_v[pl.ds(c * _CH + k * _L, _L)]
                coff = jax.lax.shift_left(
                    jax.lax.bitwise_and(ids16, jnp.int32(1)), jnp.int32(6))
                jo = j + c * _CH

                jp = jax.lax.shift_right_logical(jo, 1)
                poff = jax.lax.shift_left(
                    jax.lax.bitwise_and(jo, jnp.int32(1)), jnp.int32(6))

                def inner(f, _2):
                    vals = plsc.load_gather(buf_v, [j, coff + f])
                    plsc.store_scatter(out_v, [jp, poff + f], vals)
                    return _2
                lax.fori_loop(0, D, inner, 0, unroll=4)
                return _
            lax.fori_loop(0, _CH // _L, extract, 0)

        pltpu.sync_copy(
            out_v,
            out_hbm.at[pl.ds(pl.multiple_of(base // 2, bpw // 2), bpw // 2), :])

    return gather


def kernel(input_ids, labels, alibi, attention_mask, embed_table):
    V, D = embed_table.shape
    ids = input_ids.reshape(-1).astype(jnp.int32)
    B = ids.shape[0]
    table2 = embed_table.reshape(V // 2, 2 * D)
    hidden = _make_gather(B, D)(table2, ids)
    hidden = hidden.reshape(input_ids.shape + (D,))
    return (hidden, labels, alibi, attention_mask)


# speedup vs baseline: 1.1210x; 1.1210x over previous
import functools

import jax
import jax.numpy as jnp
from jax import lax
from jax.experimental import pallas as pl
from jax.experimental.pallas import tpu as pltpu
from jax.experimental.pallas import tpu_sc as plsc

_NC = 2
_NS = 16
_NW = _NC * _NS


@functools.lru_cache(maxsize=None)
def _make_gather(B: int, D: int):
    assert B % (8 * _NW) == 0
    bpw = B // _NW

    mesh = plsc.VectorSubcoreMesh(core_axis_name="c", subcore_axis_name="s")

    @functools.partial(
        pl.kernel,
        out_type=jax.ShapeDtypeStruct((B, D), jnp.float32),
        mesh=mesh,
        scratch_types=[
            pltpu.VMEM((bpw,), jnp.int32),
            pltpu.VMEM((bpw, D), jnp.float32),
            pltpu.SemaphoreType.DMA,
        ],
        compiler_params=pltpu.CompilerParams(use_tc_tiling_on_sc=False),
    )
    def gather(table_hbm, idx_hbm, out_hbm, idx_v, rows_v, sem):
        wid = lax.axis_index("s") * _NC + lax.axis_index("c")
        base = wid * bpw
        pltpu.sync_copy(idx_hbm.at[pl.ds(base, bpw)], idx_v)
        pltpu.async_copy(table_hbm.at[idx_v], rows_v, sem).wait()
        pltpu.sync_copy(rows_v, out_hbm.at[pl.ds(base, bpw)])

    return gather


def kernel(input_ids, labels, alibi, attention_mask, embed_table):
    ids = input_ids.reshape(-1).astype(jnp.int32)
    B = ids.shape[0]
    D = embed_table.shape[1]
    hidden = _make_gather(B, D)(embed_table, ids)
    hidden = hidden.reshape(input_ids.shape + (D,))
    return (hidden, labels, alibi, attention_mask)


# speedup vs baseline: 1.4072x; 1.2553x over previous
import functools

import jax
import jax.numpy as jnp
from jax import lax
from jax.experimental import pallas as pl
from jax.experimental.pallas import tpu as pltpu
from jax.experimental.pallas import tpu_sc as plsc

_NC = 2
_NS = 16
_NW = _NC * _NS
_L = 16
_CH = 64


@functools.lru_cache(maxsize=None)
def _make_gather(B: int, V: int, D: int):
    assert B % (8 * _NW) == 0
    bpw = B // _NW
    D2 = 2 * D

    mesh = plsc.VectorSubcoreMesh(core_axis_name="c", subcore_axis_name="s")

    @functools.partial(
        pl.kernel,
        out_type=jax.ShapeDtypeStruct((B // 2, D2), jnp.float32),
        mesh=mesh,
        scratch_types=[
            pltpu.VMEM((bpw,), jnp.int32),
            pltpu.VMEM((_CH, 8, D), jnp.float32),
            pltpu.VMEM((_CH // 2, D2), jnp.float32),
            pltpu.SemaphoreType.DMA,
        ],
        compiler_params=pltpu.CompilerParams(
            use_tc_tiling_on_sc=True, needs_layout_passes=False),
    )
    def gather(table_hbm, idx_hbm, out_hbm, ids_v, buf_v, out_v, sem):
        wid = lax.axis_index("s") * _NC + lax.axis_index("c")
        base = pl.multiple_of(wid * bpw, bpw)
        pltpu.sync_copy(idx_hbm.at[pl.ds(base, bpw)], ids_v)
        lanes = lax.iota(jnp.int32, _L)

        for c in range(bpw // _CH):
            def fire(k, _, c=c):
                v16 = ids_v[pl.ds(c * _CH + k * _L, _L)]
                for l in range(_L):
                    t = jnp.max(jnp.where(lanes == l, v16, 0))
                    r8 = pl.multiple_of(
                        jax.lax.shift_left(
                            jax.lax.shift_right_logical(t, 3), 3), 8)
                    pltpu.make_async_copy(
                        table_hbm.at[pl.ds(r8, 8), :],
                        buf_v.at[k * _L + l],
                        sem,
                    ).start()
                return _
            lax.fori_loop(0, _CH // _L, fire, 0)

            def drain(j, _):
                pltpu.make_async_copy(
                    table_hbm.at[pl.ds(0, 8), :], buf_v.at[0], sem,
                ).wait()
                return _
            lax.fori_loop(0, _CH, drain, 0, unroll=4)

            def extract(k, _, c=c):
                j = lax.iota(jnp.int32, _L) + k * _L
                ids16 = ids_v[pl.ds(c * _CH + k * _L, _L)]
                rlo = jax.lax.bitwise_and(ids16, jnp.int32(7))
                jp = jax.lax.shift_right_logical(j, 1)
                poff = jax.lax.shift_left(
                    jax.lax.bitwise_and(j, jnp.int32(1)), jnp.int32(6))

                def inner(f, _2):
                    fv = jnp.full((_L,), 0, jnp.int32) + f
                    vals = plsc.load_gather(buf_v, [j, rlo, fv])
                    plsc.store_scatter(out_v, [jp, poff + f], vals)
                    return _2
                lax.fori_loop(0, D, inner, 0, unroll=4)
                return _
            lax.fori_loop(0, _CH // _L, extract, 0)

            orow = pl.multiple_of((base + c * _CH) // 2, _CH // 2)
            pltpu.sync_copy(out_v, out_hbm.at[pl.ds(orow, _CH // 2), :])

    return gather


def kernel(input_ids, labels, alibi, attention_mask, embed_table):
    V, D = embed_table.shape
    ids = input_ids.reshape(-1).astype(jnp.int32)
    B = ids.shape[0]
    hidden = _make_gather(B, V, D)(embed_table, ids)
    hidden = hidden.reshape(input_ids.shape + (D,))
    return (hidden, labels, alibi, attention_mask)


# speedup vs baseline: 1.5527x; 1.1034x over previous
import functools

import jax
import jax.numpy as jnp
from jax import lax
from jax.experimental import pallas as pl
from jax.experimental.pallas import tpu as pltpu
from jax.experimental.pallas import tpu_sc as plsc

_NC = 2
_NS = 16
_NW = _NC * _NS
_L = 16
_CH = 32


@functools.lru_cache(maxsize=None)
def _make_gather(B: int, V: int, D: int):
    assert B % (8 * _NW) == 0
    bpw = B // _NW
    D2 = 2 * D

    mesh = plsc.VectorSubcoreMesh(core_axis_name="c", subcore_axis_name="s")

    @functools.partial(
        pl.kernel,
        out_type=jax.ShapeDtypeStruct((B // 2, D2), jnp.float32),
        mesh=mesh,
        scratch_types=[
            pltpu.VMEM((bpw,), jnp.int32),
            pltpu.VMEM((2, _CH, 8, D), jnp.float32),
            pltpu.VMEM((_CH // 2, D2), jnp.float32),
            pltpu.SemaphoreType.DMA,
            pltpu.SemaphoreType.DMA,
        ],
        compiler_params=pltpu.CompilerParams(
            use_tc_tiling_on_sc=True, needs_layout_passes=False),
    )
    def gather(table_hbm, idx_hbm, out_hbm, ids_v, buf_v, out_v, semA, semB):
        wid = lax.axis_index("s") * _NC + lax.axis_index("c")
        base = pl.multiple_of(wid * bpw, bpw)
        pltpu.sync_copy(idx_hbm.at[pl.ds(base, bpw)], ids_v)
        lanes = lax.iota(jnp.int32, _L)
        sems = (semA, semB)
        n_waves = bpw // _CH

        def fire_wave(c, slot):
            sem = sems[slot]

            def fire(k, _, c=c, slot=slot, sem=sem):
                v16 = ids_v[pl.ds(c * _CH + k * _L, _L)]
                for l in range(_L):
                    t = jnp.max(jnp.where(lanes == l, v16, 0))
                    r8 = pl.multiple_of(
                        jax.lax.shift_left(
                            jax.lax.shift_right_logical(t, 3), 3), 8)
                    pltpu.make_async_copy(
                        table_hbm.at[pl.ds(r8, 8), :],
                        buf_v.at[slot, k * _L + l],
                        sem,
                    ).start()
                return _
            lax.fori_loop(0, _CH // _L, fire, 0)

        def drain_wave(slot):
            sem = sems[slot]

            def drain(j, _, slot=slot, sem=sem):
                pltpu.make_async_copy(
                    table_hbm.at[pl.ds(0, 8), :], buf_v.at[slot, 0], sem,
                ).wait()
                return _
            lax.fori_loop(0, _CH, drain, 0, unroll=4)

        def extract_wave(c, slot):
            def extract(k, _, c=c, slot=slot):
                j = lax.iota(jnp.int32, _L) + k * _L
                ids16 = ids_v[pl.ds(c * _CH + k * _L, _L)]
                rlo = jax.lax.bitwise_and(ids16, jnp.int32(7))
                jp = jax.lax.shift_right_logical(j, 1)
                poff = jax.lax.shift_left(
                    jax.lax.bitwise_and(j, jnp.int32(1)), jnp.int32(6))
                sv = jnp.full((_L,), slot, jnp.int32)

                def inner(f, _2):
                    fv = jnp.full((_L,), 0, jnp.int32) + f
                    vals = plsc.load_gather(buf_v, [sv, j, rlo, fv])
                    plsc.store_scatter(out_v, [jp, poff + f], vals)
                    return _2
                lax.fori_loop(0, D, inner, 0, unroll=4)
                return _
            lax.fori_loop(0, _CH // _L, extract, 0)

            orow = pl.multiple_of((base + c * _CH) // 2, _CH // 2)
            pltpu.sync_copy(out_v, out_hbm.at[pl.ds(orow, _CH // 2), :])

        n_pairs = n_waves // 2
        fire_wave(0, 0)

        def pair(p, _):
            c0 = p * 2
            fire_wave(c0 + 1, 1)
            drain_wave(0)
            extract_wave(c0, 0)

            @pl.when(p + 1 < n_pairs)
            def _fire_next():
                fire_wave(c0 + 2, 0)

            drain_wave(1)
            extract_wave(c0 + 1, 1)
            return _
        lax.fori_loop(0, n_pairs, pair, 0)

    return gather


def kernel(input_ids, labels, alibi, attention_mask, embed_table):
    V, D = embed_table.shape
    ids = input_ids.reshape(-1).astype(jnp.int32)
    B = ids.shape[0]
    hidden = _make_gather(B, V, D)(embed_table, ids)
    hidden = hidden.reshape(input_ids.shape + (D,))
    return (hidden, labels, alibi, attention_mask)
